# baseline (speedup 1.0000x reference)
import functools

import jax
import jax.numpy as jnp
from jax import lax
from jax.experimental import pallas as pl
from jax.experimental.pallas import tpu as pltpu
from jax.experimental.pallas import tpu_sc as plsc

F32 = jnp.float32

_N0, _N1, _N2, _N3 = 100000, 25000, 6400, 1024
_E0, _E1, _E2 = 400000, 102400, 16384
_F_IN, _H, _C = 512, 128, 128

_HX = 144
_NC, _NS = 2, 16
_B = 128


def _mm_body(x_ref, w_ref, o_ref):
    o_ref[...] = jnp.dot(x_ref[...], w_ref[...], preferred_element_type=F32)


def _embed(x, w_t):
    blk = 2000
    grid = _N0 // blk
    return pl.pallas_call(
        _mm_body,
        grid=(grid,),
        in_specs=[
            pl.BlockSpec((blk, _F_IN), lambda i: (i, 0)),
            pl.BlockSpec((_F_IN, _H), lambda i: (0, 0)),
        ],
        out_specs=pl.BlockSpec((blk, _H), lambda i: (i, 0)),
        out_shape=jax.ShapeDtypeStruct((_N0, _H), F32),
    )(x, w_t)


def _layer0_body(hd_ref, sm_ref, dg_ref, ws_ref, wn_ref, b_ref, o_ref):
    blk = o_ref.shape[0]
    deg = jnp.maximum(dg_ref[...][:, :1], 1.0)
    neigh = sm_ref[...] / deg
    acc = (jnp.dot(hd_ref[...], ws_ref[...], preferred_element_type=F32)
           + jnp.dot(neigh, wn_ref[...], preferred_element_type=F32)
           + b_ref[...])
    acc = jnp.maximum(acc, 0.0)
    o_ref[...] = jnp.concatenate([acc, jnp.ones((blk, 16), F32)], axis=1)


def _layer1_body(hd_ref, sm_ref, ws_ref, wn_ref, b_ref, o_ref):
    blk = o_ref.shape[0]
    smx = sm_ref[...]
    deg = jnp.maximum(smx[:, _H:_H + 1], 1.0)
    neigh = smx[:, :_H] / deg
    acc = (jnp.dot(hd_ref[...][:, :_H], ws_ref[...], preferred_element_type=F32)
           + jnp.dot(neigh, wn_ref[...], preferred_element_type=F32)
           + b_ref[...])
    acc = jnp.maximum(acc, 0.0)
    o_ref[...] = jnp.concatenate([acc, jnp.ones((blk, 16), F32)], axis=1)


def _layer2_fc_body(hd_ref, sm_ref, ws_ref, wn_ref, b_ref, fw_ref, fb_ref,
                    o_ref):
    smx = sm_ref[...]
    deg = jnp.maximum(smx[:, _H:_H + 1], 1.0)
    neigh = smx[:, :_H] / deg
    acc = (jnp.dot(hd_ref[...][:, :_H], ws_ref[...], preferred_element_type=F32)
           + jnp.dot(neigh, wn_ref[...], preferred_element_type=F32)
           + b_ref[...])
    o_ref[...] = jnp.dot(acc, fw_ref[...], preferred_element_type=F32) + fb_ref[...]


def _layer0(h_prev, sums, deg, ws_t, wn_t, b, n_out, blk):
    grid = n_out // blk
    return pl.pallas_call(
        _layer0_body,
        grid=(grid,),
        in_specs=[
            pl.BlockSpec((blk, _H), lambda i: (i, 0)),
            pl.BlockSpec((blk, _H), lambda i: (i, 0)),
            pl.BlockSpec((blk, 16), lambda i: (i, 0)),
            pl.BlockSpec((_H, _H), lambda i: (0, 0)),
            pl.BlockSpec((_H, _H), lambda i: (0, 0)),
            pl.BlockSpec((1, _H), lambda i: (0, 0)),
        ],
        out_specs=pl.BlockSpec((blk, _HX), lambda i: (i, 0)),
        out_shape=jax.ShapeDtypeStruct((n_out, _HX), F32),
    )(h_prev, sums, deg, ws_t, wn_t, b)


def _layer1(h_prev, sums, ws_t, wn_t, b, n_out, blk):
    grid = n_out // blk
    return pl.pallas_call(
        _layer1_body,
        grid=(grid,),
        in_specs=[
            pl.BlockSpec((blk, _HX), lambda i: (i, 0)),
            pl.BlockSpec((blk, _HX), lambda i: (i, 0)),
            pl.BlockSpec((_H, _H), lambda i: (0, 0)),
            pl.BlockSpec((_H, _H), lambda i: (0, 0)),
            pl.BlockSpec((1, _H), lambda i: (0, 0)),
        ],
        out_specs=pl.BlockSpec((blk, _HX), lambda i: (i, 0)),
        out_shape=jax.ShapeDtypeStruct((n_out, _HX), F32),
    )(h_prev, sums, ws_t, wn_t, b)


def _layer2_fc(h_prev, sums, ws_t, wn_t, b, fw_t, fb, n_out):
    return pl.pallas_call(
        _layer2_fc_body,
        grid=(1,),
        in_specs=[
            pl.BlockSpec((n_out, _HX), lambda i: (0, 0)),
            pl.BlockSpec((n_out, _HX), lambda i: (0, 0)),
            pl.BlockSpec((_H, _H), lambda i: (0, 0)),
            pl.BlockSpec((_H, _H), lambda i: (0, 0)),
            pl.BlockSpec((1, _H), lambda i: (0, 0)),
            pl.BlockSpec((_H, _C), lambda i: (0, 0)),
            pl.BlockSpec((1, _C), lambda i: (0, 0)),
        ],
        out_specs=pl.BlockSpec((n_out, _C), lambda i: (0, 0)),
        out_shape=jax.ShapeDtypeStruct((n_out, _C), F32),
    )(h_prev, sums, ws_t, wn_t, b, fw_t, fb)


_MESH = plsc.VectorSubcoreMesh(core_axis_name="c", subcore_axis_name="s",
                               num_cores=_NC, num_subcores=_NS)


def _make_sum_agg(chunks, split, rng, alloc, zspan, trash, wout, n_out,
                  B=96, nbuf=2, name="sum", W=_H, ncopy=1):
    assert chunks % nbuf == 0 and chunks >= 2 * nbuf
    assert zspan == ncopy * alloc // 16 and zspan % 8 == 0

    @functools.partial(
        pl.kernel,
        out_type=jax.ShapeDtypeStruct((n_out, W), F32),
        mesh=_MESH,
        name=name,
        compiler_params=pltpu.CompilerParams(
            use_tc_tiling_on_sc=(W == _H)),
        scratch_types=[
            pltpu.VMEM((nbuf, B), jnp.int32),
            pltpu.VMEM((nbuf, B), jnp.int32),
            pltpu.VMEM((nbuf, B), jnp.int32),
            pltpu.VMEM((nbuf, B, W), F32),
            pltpu.VMEM((2, wout if ncopy > 1 else 8, W), F32),
            pltpu.VMEM_SHARED((ncopy * alloc, W), F32),
        ] + [pltpu.SemaphoreType.DMA] * (3 * nbuf),
    )
    def agg(h_hbm, src_hbm, dst_hbm, sums_out,
            idx_src, idx_dst, idx_loc, rows, wr, sums_sh, *sems):
        gsem, ssem, isem = sems[:nbuf], sems[nbuf:2 * nbuf], sems[2 * nbuf:]
        c = lax.axis_index("c")
        s = lax.axis_index("s")

        def zrow(i, _):
            def zcol(j, _):
                rows[0, i, pl.ds(j * 16, 16)] = jnp.zeros((16,), F32)
                return 0
            lax.fori_loop(0, W // 16, zcol, 0)
            return 0
        lax.fori_loop(0, 8, zrow, 0)

        def zshared(t, _):
            off = s * zspan + t * 8
            pltpu.sync_copy(rows.at[0].at[pl.ds(0, 8)],
                            sums_sh.at[pl.ds(off, 8)])
            return 0
        lax.fori_loop(0, zspan // 8, zshared, 0)

        plsc.subcore_barrier()

        lo = c * split
        base0 = s * chunks * B
        cbase = lax.rem(s, ncopy) * alloc if ncopy > 1 else 0

        def compute_loc(b):
            def loc16(k, _):
                d = idx_dst[b, pl.ds(k * 16, 16)]
                l = d - lo
                ok = (l >= 0) & (l < rng)
                idx_loc[b, pl.ds(k * 16, 16)] = jnp.where(ok, l, trash) + cbase
                return 0
            lax.fori_loop(0, B // 16, loc16, 0)

        def fire_gather(b):
            pltpu.async_copy(h_hbm.at[idx_src.at[b]], rows.at[b], gsem[b])

        def wait_gather(b):
            pltpu.make_async_copy(h_hbm.at[idx_src.at[b]], rows.at[b],
                                  gsem[b]).wait()

        def fire_scatter(b):
            pltpu.async_copy(rows.at[b], sums_sh.at[idx_loc.at[b]], ssem[b],
                             add=True)

        def wait_scatter(b):
            pltpu.make_async_copy(rows.at[b], sums_sh.at[idx_loc.at[b]],
                                  ssem[b]).wait()

        for b in range(nbuf):
            base = base0 + b * B
            pltpu.sync_copy(src_hbm.at[pl.ds(base, B)], idx_src.at[b])
            pltpu.sync_copy(dst_hbm.at[pl.ds(base, B)], idx_dst.at[b])
            fire_gather(b)

        def body(g, _):
            for b in range(nbuf):
                nbase = base0 + (nbuf * g + b + nbuf) * B
                wait_gather(b)
                compute_loc(b)
                fire_scatter(b)
                pltpu.async_copy(src_hbm.at[pl.ds(nbase, B)],
                                 idx_src.at[b], isem[b])
                pltpu.async_copy(dst_hbm.at[pl.ds(nbase, B)],
                                 idx_dst.at[b], isem[b])
                wait_scatter(b)
                pltpu.make_async_copy(src_hbm.at[pl.ds(nbase, B)],
                                      idx_src.at[b], isem[b]).wait()
                pltpu.make_async_copy(dst_hbm.at[pl.ds(nbase, B)],
                                      idx_dst.at[b], isem[b]).wait()
                fire_gather(b)
            return 0
        lax.fori_loop(0, chunks // nbuf - 1, body, 0)

        for b in range(nbuf):
            wait_gather(b)
            compute_loc(b)
            fire_scatter(b)
            wait_scatter(b)

        plsc.subcore_barrier()

        off = c * split + s * wout
        if ncopy == 1:
            pltpu.sync_copy(sums_sh.at[pl.ds(s * wout, wout)],
                            sums_out.at[pl.ds(off, wout)])
        else:
            pltpu.sync_copy(sums_sh.at[pl.ds(s * wout, wout)], wr.at[0])
            for k in range(1, ncopy):
                pltpu.sync_copy(sums_sh.at[pl.ds(k * alloc + s * wout, wout)],
                                wr.at[1])
                def radd(i, _):
                    def cadd(j, _):
                        wr[0, i, pl.ds(j * 16, 16)] = (
                            wr[0, i, pl.ds(j * 16, 16)]
                            + wr[1, i, pl.ds(j * 16, 16)])
                        return 0
                    lax.fori_loop(0, W // 16, cadd, 0)
                    return 0
                lax.fori_loop(0, wout, radd, 0)
            pltpu.sync_copy(wr.at[0], sums_out.at[pl.ds(off, wout)])

    return agg


def _make_deg_agg(chunks, split, rng, alloc, zspan, trash, wout, n_out,
                  name="deg"):
    assert chunks % 4 == 0 and chunks >= 8
    nbuf = 4

    @functools.partial(
        pl.kernel,
        out_type=jax.ShapeDtypeStruct((n_out, 16), F32),
        mesh=_MESH,
        name=name,
        scratch_types=[
            pltpu.VMEM((nbuf, _B), jnp.int32),
            pltpu.VMEM((nbuf, _B), jnp.int32),
            pltpu.VMEM((_B, 16), F32),
            pltpu.VMEM_SHARED((alloc, 16), F32),
        ] + [pltpu.SemaphoreType.DMA] * (2 * nbuf),
    )
    def agg(dst_hbm, deg_out, idx_dst, idx_loc, ones_b, deg_sh, *sems):
        ssem, isem = sems[:nbuf], sems[nbuf:]
        c = lax.axis_index("c")
        s = lax.axis_index("s")

        def zrow(i, _):
            ones_b[i, :] = jnp.zeros((16,), F32)
            return 0
        lax.fori_loop(0, 8, zrow, 0)

        def zshared(t, _):
            off = s * zspan + t * 8
            pltpu.sync_copy(ones_b.at[pl.ds(0, 8)], deg_sh.at[pl.ds(off, 8)])
            return 0
        lax.fori_loop(0, zspan // 8, zshared, 0)

        def fill_ones(i, _):
            ones_b[i, :] = jnp.ones((16,), F32)
            return 0
        lax.fori_loop(0, _B, fill_ones, 0)

        plsc.subcore_barrier()

        lo = c * split
        base0 = s * chunks * _B

        def compute_loc(b):
            def loc16(k, _):
                d = idx_dst[b, pl.ds(k * 16, 16)]
                l = d - lo
                ok = (l >= 0) & (l < rng)
                idx_loc[b, pl.ds(k * 16, 16)] = jnp.where(ok, l, trash)
                return 0
            lax.fori_loop(0, _B // 16, loc16, 0)

        def fire_scatter(b):
            pltpu.async_copy(ones_b, deg_sh.at[idx_loc.at[b]], ssem[b],
                             add=True)

        def wait_scatter(b):
            pltpu.make_async_copy(ones_b, deg_sh.at[idx_loc.at[b]],
                                  ssem[b]).wait()

        for b in range(nbuf):
            base = base0 + b * _B
            pltpu.sync_copy(dst_hbm.at[pl.ds(base, _B)], idx_dst.at[b])
            compute_loc(b)
            fire_scatter(b)

        def body(g, _):
            for b in range(nbuf):
                nbase = base0 + (4 * g + b + 4) * _B
                pltpu.async_copy(dst_hbm.at[pl.ds(nbase, _B)],
                                 idx_dst.at[b], isem[b])
                wait_scatter(b)
                pltpu.make_async_copy(dst_hbm.at[pl.ds(nbase, _B)],
                                      idx_dst.at[b], isem[b]).wait()
                compute_loc(b)
                fire_scatter(b)
            return 0
        lax.fori_loop(0, chunks // 4 - 1, body, 0)

        for b in range(nbuf):
            wait_scatter(b)

        plsc.subcore_barrier()

        off = c * split + s * wout
        pltpu.sync_copy(deg_sh.at[pl.ds(s * wout, wout)],
                        deg_out.at[pl.ds(off, wout)])

    return agg


_CFG0 = (12544, 12544, 12672, 792, 12600, 784, 25088)
_CFG1 = (3200, 3200, 3328, 208, 3264, 200, _N2)
_CFG2 = (512, 512, 640, 40, 576, 32, _N3)
_SUM0 = _make_sum_agg(262, *_CFG0, name="sum0")
_DEG0 = _make_deg_agg(196, *_CFG0, name="deg0")
_SUM1 = _make_sum_agg(68, *_CFG1, name="sum1", W=_HX)
_SUM2 = _make_sum_agg(12, 512, 512, 640, 320, 576, 32, _N3, name="sum2",
                      W=_HX, ncopy=8)


@jax.jit
def kernel(x, src0, dst0, src1, dst1, src2, dst2, embed_W,
           Wself0, Wneigh0, b0, Wself1, Wneigh1, b1, Wself2, Wneigh2, b2,
           fcW, fcb):
    def _pad_edges(src, dst, total, dump):
        pad = total - src.shape[0]
        return (jnp.concatenate([src, jnp.zeros((pad,), jnp.int32)]),
                jnp.concatenate([dst, jnp.full((pad,), dump, jnp.int32)]))

    src0p, dst0p = _pad_edges(src0, dst0, 402432, _N1)
    src1p, dst1p = _pad_edges(src1, dst1, 106496, _N2)
    src2p, dst2p = _pad_edges(src2, dst2, 18432, _N3)

    d0 = _DEG0(dst0p)

    h0 = _embed(x, embed_W.T)

    s0 = _SUM0(h0, src0p, dst0p)
    h1 = _layer0(h0, s0, d0, Wself0.T, Wneigh0.T, b0.reshape(1, -1),
                 n_out=25088, blk=784)
    s1 = _SUM1(h1, src1p, dst1p)
    h2 = _layer1(h1, s1, Wself1.T, Wneigh1.T, b1.reshape(1, -1),
                 n_out=_N2, blk=800)

    s2 = _SUM2(h2, src2p, dst2p)
    return _layer2_fc(h2, s2, Wself2.T, Wneigh2.T, b2.reshape(1, -1),
                      fcW.T, fcb.reshape(1, -1), n_out=_N3)


# speedup vs baseline: 1.1454x; 1.1454x over previous
import functools

import jax
import jax.numpy as jnp
from jax import lax
from jax.experimental import pallas as pl
from jax.experimental.pallas import tpu as pltpu
from jax.experimental.pallas import tpu_sc as plsc

F32 = jnp.float32

_N0, _N1, _N2, _N3 = 100000, 25000, 6400, 1024
_E0, _E1, _E2 = 400000, 102400, 16384
_F_IN, _H, _C = 512, 128, 128

_HX = 144
_NC, _NS = 2, 16
_B = 128


def _mm_body(x_ref, w_ref, o_ref):
    o_ref[...] = jnp.dot(x_ref[...], w_ref[...], preferred_element_type=F32)


def _embed(x, w_t):
    blk = 2000
    grid = _N0 // blk
    return pl.pallas_call(
        _mm_body,
        grid=(grid,),
        in_specs=[
            pl.BlockSpec((blk, _F_IN), lambda i: (i, 0)),
            pl.BlockSpec((_F_IN, _H), lambda i: (0, 0)),
        ],
        out_specs=pl.BlockSpec((blk, _H), lambda i: (i, 0)),
        out_shape=jax.ShapeDtypeStruct((_N0, _H), F32),
    )(x, w_t)


def _layer0_body(hd_ref, sm_ref, dga_ref, dgb_ref, ws_ref, wn_ref, b_ref,
                 o_ref):
    blk = o_ref.shape[0]
    dg = dga_ref[...] + dgb_ref[...]
    dg = dg * 0.5
    deg = jnp.maximum(dg[:, :1], 1.0)
    neigh = sm_ref[...] / deg
    acc = (jnp.dot(hd_ref[...], ws_ref[...], preferred_element_type=F32)
           + jnp.dot(neigh, wn_ref[...], preferred_element_type=F32)
           + b_ref[...])
    acc = jnp.maximum(acc, 0.0)
    o_ref[...] = jnp.concatenate([acc, jnp.ones((blk, 16), F32)], axis=1)


def _layer1_body(hd_ref, sma_ref, smb_ref, ws_ref, wn_ref, b_ref, o_ref):
    blk = o_ref.shape[0]
    smx = sma_ref[...] + smb_ref[...]
    deg = jnp.maximum(smx[:, _H:_H + 1], 1.0)
    neigh = smx[:, :_H] / deg
    acc = (jnp.dot(hd_ref[...][:, :_H], ws_ref[...], preferred_element_type=F32)
           + jnp.dot(neigh, wn_ref[...], preferred_element_type=F32)
           + b_ref[...])
    acc = jnp.maximum(acc, 0.0)
    o_ref[...] = jnp.concatenate([acc, jnp.ones((blk, 16), F32)], axis=1)


def _layer2_fc_body(hd_ref, sma_ref, smb_ref, ws_ref, wn_ref, b_ref,
                    fw_ref, fb_ref, o_ref):
    smx = sma_ref[...] + smb_ref[...]
    deg = jnp.maximum(smx[:, _H:_H + 1], 1.0)
    neigh = smx[:, :_H] / deg
    acc = (jnp.dot(hd_ref[...][:, :_H], ws_ref[...], preferred_element_type=F32)
           + jnp.dot(neigh, wn_ref[...], preferred_element_type=F32)
           + b_ref[...])
    o_ref[...] = jnp.dot(acc, fw_ref[...], preferred_element_type=F32) + fb_ref[...]


def _layer0(h_prev, sums, deg, ws_t, wn_t, b, n_out, blk):
    grid = n_out // blk
    return pl.pallas_call(
        _layer0_body,
        grid=(grid,),
        in_specs=[
            pl.BlockSpec((blk, _H), lambda i: (i, 0)),
            pl.BlockSpec((blk, _H), lambda i: (i, 0)),
            pl.BlockSpec((blk, 16), lambda i: (i, 0)),
            pl.BlockSpec((blk, 16), lambda i: (i, 0)),
            pl.BlockSpec((_H, _H), lambda i: (0, 0)),
            pl.BlockSpec((_H, _H), lambda i: (0, 0)),
            pl.BlockSpec((1, _H), lambda i: (0, 0)),
        ],
        out_specs=pl.BlockSpec((blk, _HX), lambda i: (i, 0)),
        out_shape=jax.ShapeDtypeStruct((n_out, _HX), F32),
    )(h_prev, sums, deg, deg, ws_t, wn_t, b)


def _layer1(h_prev, sums, ws_t, wn_t, b, n_out, blk):
    grid = n_out // blk
    return pl.pallas_call(
        _layer1_body,
        grid=(grid,),
        in_specs=[
            pl.BlockSpec((blk, _HX), lambda i: (i, 0)),
            pl.BlockSpec((blk, _HX), lambda i: (i, 0)),
            pl.BlockSpec((blk, _HX), lambda i: (i + grid, 0)),
            pl.BlockSpec((_H, _H), lambda i: (0, 0)),
            pl.BlockSpec((_H, _H), lambda i: (0, 0)),
            pl.BlockSpec((1, _H), lambda i: (0, 0)),
        ],
        out_specs=pl.BlockSpec((blk, _HX), lambda i: (i, 0)),
        out_shape=jax.ShapeDtypeStruct((n_out, _HX), F32),
    )(h_prev, sums, sums, ws_t, wn_t, b)


def _layer2_fc(h_prev, sums, ws_t, wn_t, b, fw_t, fb, n_out):
    return pl.pallas_call(
        _layer2_fc_body,
        grid=(1,),
        in_specs=[
            pl.BlockSpec((n_out, _HX), lambda i: (0, 0)),
            pl.BlockSpec((n_out, _HX), lambda i: (0, 0)),
            pl.BlockSpec((n_out, _HX), lambda i: (1, 0)),
            pl.BlockSpec((_H, _H), lambda i: (0, 0)),
            pl.BlockSpec((_H, _H), lambda i: (0, 0)),
            pl.BlockSpec((1, _H), lambda i: (0, 0)),
            pl.BlockSpec((_H, _C), lambda i: (0, 0)),
            pl.BlockSpec((1, _C), lambda i: (0, 0)),
        ],
        out_specs=pl.BlockSpec((n_out, _C), lambda i: (0, 0)),
        out_shape=jax.ShapeDtypeStruct((n_out, _C), F32),
    )(h_prev, sums, sums, ws_t, wn_t, b, fw_t, fb)


_MESH = plsc.VectorSubcoreMesh(core_axis_name="c", subcore_axis_name="s",
                               num_cores=_NC, num_subcores=_NS)


def _make_sum_agg(chunks, split, rng, alloc, zspan, trash, wout, n_out,
                  B=96, nbuf=2, name="sum", W=_H, ncopy=1,
                  edge_split=False):
    assert chunks % nbuf == 0 and chunks >= 2 * nbuf
    assert zspan == ncopy * alloc // 16 and zspan % 8 == 0
    out_shape = (_NC * n_out, W) if edge_split else (n_out, W)

    @functools.partial(
        pl.kernel,
        out_type=jax.ShapeDtypeStruct(out_shape, F32),
        mesh=_MESH,
        name=name,
        compiler_params=pltpu.CompilerParams(
            use_tc_tiling_on_sc=(W == _H)),
        scratch_types=[
            pltpu.VMEM((nbuf, B), jnp.int32),
            pltpu.VMEM((nbuf, B), jnp.int32),
            pltpu.VMEM((nbuf, B), jnp.int32),
            pltpu.VMEM((nbuf, B, W), F32),
            pltpu.VMEM((2, wout if ncopy > 1 else 8, W), F32),
            pltpu.VMEM_SHARED((ncopy * alloc, W), F32),
        ] + [pltpu.SemaphoreType.DMA] * (3 * nbuf),
    )
    def agg(h_hbm, src_hbm, dst_hbm, sums_out,
            idx_src, idx_dst, idx_loc, rows, wr, sums_sh, *sems):
        gsem, ssem, isem = sems[:nbuf], sems[nbuf:2 * nbuf], sems[2 * nbuf:]
        c = lax.axis_index("c")
        s = lax.axis_index("s")

        def zrow(i, _):
            def zcol(j, _):
                rows[0, i, pl.ds(j * 16, 16)] = jnp.zeros((16,), F32)
                return 0
            lax.fori_loop(0, W // 16, zcol, 0)
            return 0
        lax.fori_loop(0, 8, zrow, 0)

        def zshared(t, _):
            off = s * zspan + t * 8
            pltpu.sync_copy(rows.at[0].at[pl.ds(0, 8)],
                            sums_sh.at[pl.ds(off, 8)])
            return 0
        lax.fori_loop(0, zspan // 8, zshared, 0)

        plsc.subcore_barrier()

        if edge_split:
            lo = 0
            base0 = (c * _NS + s) * chunks * B
        else:
            lo = c * split
            base0 = s * chunks * B
        cbase = lax.rem(s, ncopy) * alloc if ncopy > 1 else 0

        def compute_loc(b):
            def loc16(k, _):
                d = idx_dst[b, pl.ds(k * 16, 16)]
                l = d - lo
                ok = (l >= 0) & (l < rng)
                idx_loc[b, pl.ds(k * 16, 16)] = jnp.where(ok, l, trash) + cbase
                return 0
            lax.fori_loop(0, B // 16, loc16, 0)

        def fire_gather(b):
            pltpu.async_copy(h_hbm.at[idx_src.at[b]], rows.at[b], gsem[b])

        def wait_gather(b):
            pltpu.make_async_copy(h_hbm.at[idx_src.at[b]], rows.at[b],
                                  gsem[b]).wait()

        def fire_scatter(b):
            pltpu.async_copy(rows.at[b], sums_sh.at[idx_loc.at[b]], ssem[b],
                             add=True)

        def wait_scatter(b):
            pltpu.make_async_copy(rows.at[b], sums_sh.at[idx_loc.at[b]],
                                  ssem[b]).wait()

        for b in range(nbuf):
            base = base0 + b * B
            pltpu.sync_copy(src_hbm.at[pl.ds(base, B)], idx_src.at[b])
            pltpu.sync_copy(dst_hbm.at[pl.ds(base, B)], idx_dst.at[b])
            fire_gather(b)

        def body(g, _):
            for b in range(nbuf):
                nbase = base0 + (nbuf * g + b + nbuf) * B
                wait_gather(b)
                compute_loc(b)
                fire_scatter(b)
                pltpu.async_copy(src_hbm.at[pl.ds(nbase, B)],
                                 idx_src.at[b], isem[b])
                pltpu.async_copy(dst_hbm.at[pl.ds(nbase, B)],
                                 idx_dst.at[b], isem[b])
                wait_scatter(b)
                pltpu.make_async_copy(src_hbm.at[pl.ds(nbase, B)],
                                      idx_src.at[b], isem[b]).wait()
                pltpu.make_async_copy(dst_hbm.at[pl.ds(nbase, B)],
                                      idx_dst.at[b], isem[b]).wait()
                fire_gather(b)
            return 0
        lax.fori_loop(0, chunks // nbuf - 1, body, 0)

        for b in range(nbuf):
            wait_gather(b)
            compute_loc(b)
            fire_scatter(b)
            wait_scatter(b)

        plsc.subcore_barrier()

        if edge_split:
            dst_ref = sums_out.at[pl.ds(c * n_out + s * wout, wout)]
        else:
            dst_ref = sums_out.at[pl.ds(c * split + s * wout, wout)]
        if ncopy == 1:
            pltpu.sync_copy(sums_sh.at[pl.ds(s * wout, wout)], dst_ref)
        else:
            pltpu.sync_copy(sums_sh.at[pl.ds(s * wout, wout)], wr.at[0])
            for k in range(1, ncopy):
                pltpu.sync_copy(sums_sh.at[pl.ds(k * alloc + s * wout, wout)],
                                wr.at[1])
                def radd(i, _):
                    def cadd(j, _):
                        wr[0, i, pl.ds(j * 16, 16)] = (
                            wr[0, i, pl.ds(j * 16, 16)]
                            + wr[1, i, pl.ds(j * 16, 16)])
                        return 0
                    lax.fori_loop(0, W // 16, cadd, 0)
                    return 0
                lax.fori_loop(0, wout, radd, 0)
            pltpu.sync_copy(wr.at[0], dst_ref)

    return agg


def _make_deg_agg(chunks, split, rng, alloc, zspan, trash, wout, n_out,
                  name="deg", edge_split=False):
    assert chunks % 4 == 0 and chunks >= 8
    nbuf = 4
    out_shape = (_NC * n_out, 16) if edge_split else (n_out, 16)

    @functools.partial(
        pl.kernel,
        out_type=jax.ShapeDtypeStruct(out_shape, F32),
        mesh=_MESH,
        name=name,
        scratch_types=[
            pltpu.VMEM((nbuf, _B), jnp.int32),
            pltpu.VMEM((nbuf, _B), jnp.int32),
            pltpu.VMEM((_B, 16), F32),
            pltpu.VMEM_SHARED((alloc, 16), F32),
        ] + [pltpu.SemaphoreType.DMA] * (2 * nbuf),
    )
    def agg(dst_hbm, deg_out, idx_dst, idx_loc, ones_b, deg_sh, *sems):
        ssem, isem = sems[:nbuf], sems[nbuf:]
        c = lax.axis_index("c")
        s = lax.axis_index("s")

        def zrow(i, _):
            ones_b[i, :] = jnp.zeros((16,), F32)
            return 0
        lax.fori_loop(0, 8, zrow, 0)

        def zshared(t, _):
            off = s * zspan + t * 8
            pltpu.sync_copy(ones_b.at[pl.ds(0, 8)], deg_sh.at[pl.ds(off, 8)])
            return 0
        lax.fori_loop(0, zspan // 8, zshared, 0)

        def fill_ones(i, _):
            ones_b[i, :] = jnp.ones((16,), F32)
            return 0
        lax.fori_loop(0, _B, fill_ones, 0)

        plsc.subcore_barrier()

        if edge_split:
            lo = 0
            base0 = (c * _NS + s) * chunks * _B
        else:
            lo = c * split
            base0 = s * chunks * _B

        def compute_loc(b):
            def loc16(k, _):
                d = idx_dst[b, pl.ds(k * 16, 16)]
                l = d - lo
                ok = (l >= 0) & (l < rng)
                idx_loc[b, pl.ds(k * 16, 16)] = jnp.where(ok, l, trash)
                return 0
            lax.fori_loop(0, _B // 16, loc16, 0)

        def fire_scatter(b):
            pltpu.async_copy(ones_b, deg_sh.at[idx_loc.at[b]], ssem[b],
                             add=True)

        def wait_scatter(b):
            pltpu.make_async_copy(ones_b, deg_sh.at[idx_loc.at[b]],
                                  ssem[b]).wait()

        for b in range(nbuf):
            base = base0 + b * _B
            pltpu.sync_copy(dst_hbm.at[pl.ds(base, _B)], idx_dst.at[b])
            compute_loc(b)
            fire_scatter(b)

        def body(g, _):
            for b in range(nbuf):
                nbase = base0 + (4 * g + b + 4) * _B
                pltpu.async_copy(dst_hbm.at[pl.ds(nbase, _B)],
                                 idx_dst.at[b], isem[b])
                wait_scatter(b)
                pltpu.make_async_copy(dst_hbm.at[pl.ds(nbase, _B)],
                                      idx_dst.at[b], isem[b]).wait()
                compute_loc(b)
                fire_scatter(b)
            return 0
        lax.fori_loop(0, chunks // 4 - 1, body, 0)

        for b in range(nbuf):
            wait_scatter(b)

        plsc.subcore_barrier()

        if edge_split:
            off = c * n_out + s * wout
        else:
            off = c * split + s * wout
        pltpu.sync_copy(deg_sh.at[pl.ds(s * wout, wout)],
                        deg_out.at[pl.ds(off, wout)])

    return agg


_SUM0 = _make_sum_agg(262, 12544, 12544, 12672, 792, 12600, 784, 25088,
                      name="sum0")
_DEG0 = _make_deg_agg(196, 12544, 12544, 12672, 792, 12600, 784, 25088,
                      name="deg0")
_SUM1 = _make_sum_agg(34, 0, _N2, 6528, 408, 6464, 400, _N2,
                      name="sum1", W=_HX, edge_split=True)
_SUM2 = _make_sum_agg(6, 0, _N3, 1152, 288, 1088, 64, _N3,
                      name="sum2", W=_HX, ncopy=4, edge_split=True)


@jax.jit
def kernel(x, src0, dst0, src1, dst1, src2, dst2, embed_W,
           Wself0, Wneigh0, b0, Wself1, Wneigh1, b1, Wself2, Wneigh2, b2,
           fcW, fcb):
    def _pad_edges(src, dst, total, dump):
        pad = total - src.shape[0]
        return (jnp.concatenate([src, jnp.zeros((pad,), jnp.int32)]),
                jnp.concatenate([dst, jnp.full((pad,), dump, jnp.int32)]))

    src0p, dst0p = _pad_edges(src0, dst0, 409600, _N1)
    src1p, dst1p = _pad_edges(src1, dst1, 104448, _N2)
    src2p, dst2p = _pad_edges(src2, dst2, 18432, _N3)

    d0 = _DEG0(dst0p)

    h0 = _embed(x, embed_W.T)

    s0 = _SUM0(h0, src0p, dst0p)
    h1 = _layer0(h0, s0, d0, Wself0.T, Wneigh0.T, b0.reshape(1, -1),
                 n_out=25088, blk=784)
    s1 = _SUM1(h1, src1p, dst1p)
    h2 = _layer1(h1, s1, Wself1.T, Wneigh1.T, b1.reshape(1, -1),
                 n_out=_N2, blk=800)

    s2 = _SUM2(h2, src2p, dst2p)
    return _layer2_fc(h2, s2, Wself2.T, Wneigh2.T, b2.reshape(1, -1),
                      fcW.T, fcb.reshape(1, -1), n_out=_N3)


# speedup vs baseline: 1.1734x; 1.0244x over previous
import functools

import jax
import jax.numpy as jnp
from jax import lax
from jax.experimental import pallas as pl
from jax.experimental.pallas import tpu as pltpu
from jax.experimental.pallas import tpu_sc as plsc

F32 = jnp.float32

_N0, _N1, _N2, _N3 = 100000, 25000, 6400, 1024
_E0, _E1, _E2 = 400000, 102400, 16384
_F_IN, _H, _C = 512, 128, 128

_HX = 144
_NC, _NS = 2, 16
_B = 128


def _mm_body(x_ref, w_ref, o_ref):
    o_ref[...] = jnp.dot(x_ref[...], w_ref[...], preferred_element_type=F32)


def _embed(x, w_t):
    blk = 2000
    grid = _N0 // blk
    return pl.pallas_call(
        _mm_body,
        grid=(grid,),
        in_specs=[
            pl.BlockSpec((blk, _F_IN), lambda i: (i, 0)),
            pl.BlockSpec((_F_IN, _H), lambda i: (0, 0)),
        ],
        out_specs=pl.BlockSpec((blk, _H), lambda i: (i, 0)),
        out_shape=jax.ShapeDtypeStruct((_N0, _H), F32),
    )(x, w_t)


def _layer0_body(hd_ref, sm_ref, dg_ref, ws_ref, wn_ref, b_ref, o_ref):
    deg = jnp.maximum(dg_ref[...][:, :1], 1.0)
    neigh = sm_ref[...] / deg
    acc = (jnp.dot(hd_ref[...], ws_ref[...], preferred_element_type=F32)
           + jnp.dot(neigh, wn_ref[...], preferred_element_type=F32)
           + b_ref[...])
    o_ref[...] = jnp.maximum(acc, 0.0)


def _layer12_body(hd_ref, sma_ref, smb_ref, dga_ref, dgb_ref,
                  ws_ref, wn_ref, b_ref, o_ref):
    dg = dga_ref[...] + dgb_ref[...]
    deg = jnp.maximum(dg[:, :1], 1.0)
    neigh = (sma_ref[...] + smb_ref[...]) / deg
    acc = (jnp.dot(hd_ref[...], ws_ref[...], preferred_element_type=F32)
           + jnp.dot(neigh, wn_ref[...], preferred_element_type=F32)
           + b_ref[...])
    o_ref[...] = jnp.maximum(acc, 0.0)


def _layer2_fc_body(hd_ref, sma_ref, smb_ref, dg_ref,
                    ws_ref, wn_ref, b_ref, fw_ref, fb_ref, o_ref):
    deg = jnp.maximum(dg_ref[...][:, :1], 1.0)
    neigh = (sma_ref[...] + smb_ref[...]) / deg
    acc = (jnp.dot(hd_ref[...], ws_ref[...], preferred_element_type=F32)
           + jnp.dot(neigh, wn_ref[...], preferred_element_type=F32)
           + b_ref[...])
    o_ref[...] = jnp.dot(acc, fw_ref[...], preferred_element_type=F32) + fb_ref[...]


def _layer0(h_prev, sums, deg, ws_t, wn_t, b, n_out, blk):
    grid = n_out // blk
    return pl.pallas_call(
        _layer0_body,
        grid=(grid,),
        in_specs=[
            pl.BlockSpec((blk, _H), lambda i: (i, 0)),
            pl.BlockSpec((blk, _H), lambda i: (i, 0)),
            pl.BlockSpec((blk, 16), lambda i: (i, 0)),
            pl.BlockSpec((_H, _H), lambda i: (0, 0)),
            pl.BlockSpec((_H, _H), lambda i: (0, 0)),
            pl.BlockSpec((1, _H), lambda i: (0, 0)),
        ],
        out_specs=pl.BlockSpec((blk, _H), lambda i: (i, 0)),
        out_shape=jax.ShapeDtypeStruct((n_out, _H), F32),
    )(h_prev, sums, deg, ws_t, wn_t, b)


def _layer1(h_prev, sums, deg, ws_t, wn_t, b, n_out, blk):
    grid = n_out // blk
    return pl.pallas_call(
        _layer12_body,
        grid=(grid,),
        in_specs=[
            pl.BlockSpec((blk, _H), lambda i: (i, 0)),
            pl.BlockSpec((blk, _H), lambda i: (i, 0)),
            pl.BlockSpec((blk, _H), lambda i: (i, 0)),
            pl.BlockSpec((blk, 16), lambda i: (i, 0)),
            pl.BlockSpec((blk, 16), lambda i: (i, 0)),
            pl.BlockSpec((_H, _H), lambda i: (0, 0)),
            pl.BlockSpec((_H, _H), lambda i: (0, 0)),
            pl.BlockSpec((1, _H), lambda i: (0, 0)),
        ],
        out_specs=pl.BlockSpec((blk, _H), lambda i: (i, 0)),
        out_shape=jax.ShapeDtypeStruct((n_out, _H), F32),
    )(h_prev, sums[:n_out], sums[n_out:], deg[:n_out], deg[n_out:],
      ws_t, wn_t, b)


def _layer2_fc(h_prev, sums, deg, ws_t, wn_t, b, fw_t, fb, n_out):
    return pl.pallas_call(
        _layer2_fc_body,
        grid=(1,),
        in_specs=[
            pl.BlockSpec((n_out, _H), lambda i: (0, 0)),
            pl.BlockSpec((n_out, _H), lambda i: (0, 0)),
            pl.BlockSpec((n_out, _H), lambda i: (0, 0)),
            pl.BlockSpec((n_out, 16), lambda i: (0, 0)),
            pl.BlockSpec((_H, _H), lambda i: (0, 0)),
            pl.BlockSpec((_H, _H), lambda i: (0, 0)),
            pl.BlockSpec((1, _H), lambda i: (0, 0)),
            pl.BlockSpec((_H, _C), lambda i: (0, 0)),
            pl.BlockSpec((1, _C), lambda i: (0, 0)),
        ],
        out_specs=pl.BlockSpec((n_out, _C), lambda i: (0, 0)),
        out_shape=jax.ShapeDtypeStruct((n_out, _C), F32),
    )(h_prev, sums[:n_out], sums[n_out:], deg, ws_t, wn_t, b, fw_t, fb)


_MESH = plsc.VectorSubcoreMesh(core_axis_name="c", subcore_axis_name="s",
                               num_cores=_NC, num_subcores=_NS)


def _make_sum_agg(chunks, split, rng, alloc, zspan, trash, wout, n_out,
                  B=96, nbuf=2, name="sum", W=_H, ncopy=1,
                  edge_split=False):
    assert chunks % nbuf == 0 and chunks >= 2 * nbuf
    assert zspan == ncopy * alloc // 16 and zspan % 8 == 0
    out_shape = (_NC * n_out, W) if edge_split else (n_out, W)

    @functools.partial(
        pl.kernel,
        out_type=jax.ShapeDtypeStruct(out_shape, F32),
        mesh=_MESH,
        name=name,
        compiler_params=pltpu.CompilerParams(
            use_tc_tiling_on_sc=(W == _H)),
        scratch_types=[
            pltpu.VMEM((nbuf, B), jnp.int32),
            pltpu.VMEM((nbuf, B), jnp.int32),
            pltpu.VMEM((nbuf, B), jnp.int32),
            pltpu.VMEM((nbuf, B, W), F32),
            pltpu.VMEM((2, wout if ncopy > 1 else 8, W), F32),
            pltpu.VMEM_SHARED((ncopy * alloc, W), F32),
        ] + [pltpu.SemaphoreType.DMA] * (3 * nbuf),
    )
    def agg(h_hbm, src_hbm, dst_hbm, sums_out,
            idx_src, idx_dst, idx_loc, rows, wr, sums_sh, *sems):
        gsem, ssem, isem = sems[:nbuf], sems[nbuf:2 * nbuf], sems[2 * nbuf:]
        c = lax.axis_index("c")
        s = lax.axis_index("s")

        def zrow(i, _):
            def zcol(j, _):
                rows[0, i, pl.ds(j * 16, 16)] = jnp.zeros((16,), F32)
                return 0
            lax.fori_loop(0, W // 16, zcol, 0)
            return 0
        lax.fori_loop(0, 8, zrow, 0)

        def zshared(t, _):
            off = s * zspan + t * 8
            pltpu.sync_copy(rows.at[0].at[pl.ds(0, 8)],
                            sums_sh.at[pl.ds(off, 8)])
            return 0
        lax.fori_loop(0, zspan // 8, zshared, 0)

        plsc.subcore_barrier()

        if edge_split:
            lo = 0
            base0 = (c * _NS + s) * chunks * B
        else:
            lo = c * split
            base0 = s * chunks * B
        cbase = lax.rem(s, ncopy) * alloc if ncopy > 1 else 0

        def compute_loc(b):
            def loc16(k, _):
                d = idx_dst[b, pl.ds(k * 16, 16)]
                l = d - lo
                ok = (l >= 0) & (l < rng)
                idx_loc[b, pl.ds(k * 16, 16)] = jnp.where(ok, l, trash) + cbase
                return 0
            lax.fori_loop(0, B // 16, loc16, 0)

        def fire_gather(b):
            pltpu.async_copy(h_hbm.at[idx_src.at[b]], rows.at[b], gsem[b])

        def wait_gather(b):
            pltpu.make_async_copy(h_hbm.at[idx_src.at[b]], rows.at[b],
                                  gsem[b]).wait()

        def fire_scatter(b):
            pltpu.async_copy(rows.at[b], sums_sh.at[idx_loc.at[b]], ssem[b],
                             add=True)

        def wait_scatter(b):
            pltpu.make_async_copy(rows.at[b], sums_sh.at[idx_loc.at[b]],
                                  ssem[b]).wait()

        for b in range(nbuf):
            base = base0 + b * B
            pltpu.sync_copy(src_hbm.at[pl.ds(base, B)], idx_src.at[b])
            pltpu.sync_copy(dst_hbm.at[pl.ds(base, B)], idx_dst.at[b])
            fire_gather(b)

        def body(g, _):
            for b in range(nbuf):
                nbase = base0 + (nbuf * g + b + nbuf) * B
                wait_gather(b)
                compute_loc(b)
                fire_scatter(b)
                pltpu.async_copy(src_hbm.at[pl.ds(nbase, B)],
                                 idx_src.at[b], isem[b])
                pltpu.async_copy(dst_hbm.at[pl.ds(nbase, B)],
                                 idx_dst.at[b], isem[b])
                wait_scatter(b)
                pltpu.make_async_copy(src_hbm.at[pl.ds(nbase, B)],
                                      idx_src.at[b], isem[b]).wait()
                pltpu.make_async_copy(dst_hbm.at[pl.ds(nbase, B)],
                                      idx_dst.at[b], isem[b]).wait()
                fire_gather(b)
            return 0
        lax.fori_loop(0, chunks // nbuf - 1, body, 0)

        for b in range(nbuf):
            wait_gather(b)
            compute_loc(b)
            fire_scatter(b)
            wait_scatter(b)

        plsc.subcore_barrier()

        if edge_split:
            dst_ref = sums_out.at[pl.ds(c * n_out + s * wout, wout)]
        else:
            dst_ref = sums_out.at[pl.ds(c * split + s * wout, wout)]
        if ncopy == 1:
            pltpu.sync_copy(sums_sh.at[pl.ds(s * wout, wout)], dst_ref)
        else:
            pltpu.sync_copy(sums_sh.at[pl.ds(s * wout, wout)], wr.at[0])
            for k in range(1, ncopy):
                pltpu.sync_copy(sums_sh.at[pl.ds(k * alloc + s * wout, wout)],
                                wr.at[1])
                def radd(i, _):
                    def cadd(j, _):
                        wr[0, i, pl.ds(j * 16, 16)] = (
                            wr[0, i, pl.ds(j * 16, 16)]
                            + wr[1, i, pl.ds(j * 16, 16)])
                        return 0
                    lax.fori_loop(0, W // 16, cadd, 0)
                    return 0
                lax.fori_loop(0, wout, radd, 0)
            pltpu.sync_copy(wr.at[0], dst_ref)

    return agg


def _make_deg_agg(chunks, split, rng, alloc, zspan, trash, wout, n_out,
                  name="deg", edge_split=False):
    assert chunks % 4 == 0 and chunks >= 8
    nbuf = 4
    out_shape = (_NC * n_out, 16) if edge_split else (n_out, 16)

    @functools.partial(
        pl.kernel,
        out_type=jax.ShapeDtypeStruct(out_shape, F32),
        mesh=_MESH,
        name=name,
        scratch_types=[
            pltpu.VMEM((nbuf, _B), jnp.int32),
            pltpu.VMEM((nbuf, _B), jnp.int32),
            pltpu.VMEM((_B, 16), F32),
            pltpu.VMEM_SHARED((alloc, 16), F32),
        ] + [pltpu.SemaphoreType.DMA] * (2 * nbuf),
    )
    def agg(dst_hbm, deg_out, idx_dst, idx_loc, ones_b, deg_sh, *sems):
        ssem, isem = sems[:nbuf], sems[nbuf:]
        c = lax.axis_index("c")
        s = lax.axis_index("s")

        def zrow(i, _):
            ones_b[i, :] = jnp.zeros((16,), F32)
            return 0
        lax.fori_loop(0, 8, zrow, 0)

        def zshared(t, _):
            off = s * zspan + t * 8
            pltpu.sync_copy(ones_b.at[pl.ds(0, 8)], deg_sh.at[pl.ds(off, 8)])
            return 0
        lax.fori_loop(0, zspan // 8, zshared, 0)

        def fill_ones(i, _):
            ones_b[i, :] = jnp.ones((16,), F32)
            return 0
        lax.fori_loop(0, _B, fill_ones, 0)

        plsc.subcore_barrier()

        if edge_split:
            lo = 0
            base0 = (c * _NS + s) * chunks * _B
        else:
            lo = c * split
            base0 = s * chunks * _B

        def compute_loc(b):
            def loc16(k, _):
                d = idx_dst[b, pl.ds(k * 16, 16)]
                l = d - lo
                ok = (l >= 0) & (l < rng)
                idx_loc[b, pl.ds(k * 16, 16)] = jnp.where(ok, l, trash)
                return 0
            lax.fori_loop(0, _B // 16, loc16, 0)

        def fire_scatter(b):
            pltpu.async_copy(ones_b, deg_sh.at[idx_loc.at[b]], ssem[b],
                             add=True)

        def wait_scatter(b):
            pltpu.make_async_copy(ones_b, deg_sh.at[idx_loc.at[b]],
                                  ssem[b]).wait()

        for b in range(nbuf):
            base = base0 + b * _B
            pltpu.sync_copy(dst_hbm.at[pl.ds(base, _B)], idx_dst.at[b])
            compute_loc(b)
            fire_scatter(b)

        def body(g, _):
            for b in range(nbuf):
                nbase = base0 + (4 * g + b + 4) * _B
                pltpu.async_copy(dst_hbm.at[pl.ds(nbase, _B)],
                                 idx_dst.at[b], isem[b])
                wait_scatter(b)
                pltpu.make_async_copy(dst_hbm.at[pl.ds(nbase, _B)],
                                      idx_dst.at[b], isem[b]).wait()
                compute_loc(b)
                fire_scatter(b)
            return 0
        lax.fori_loop(0, chunks // 4 - 1, body, 0)

        for b in range(nbuf):
            wait_scatter(b)

        plsc.subcore_barrier()

        if edge_split:
            off = c * n_out + s * wout
        else:
            off = c * split + s * wout
        pltpu.sync_copy(deg_sh.at[pl.ds(s * wout, wout)],
                        deg_out.at[pl.ds(off, wout)])

    return agg


_SUM0 = _make_sum_agg(262, 12544, 12544, 12672, 792, 12600, 784, 25088,
                      name="sum0")
_DEG0 = _make_deg_agg(196, 12544, 12544, 12672, 792, 12600, 784, 25088,
                      name="deg0")
_SUM1 = _make_sum_agg(34, 0, _N2, 6528, 408, 6464, 400, _N2,
                      name="sum1", edge_split=True)
_SUM2 = _make_sum_agg(6, 0, _N3, 1152, 288, 1088, 64, _N3,
                      name="sum2", ncopy=4, edge_split=True)
_DEG1 = _make_deg_agg(28, 0, _N2, 6528, 408, 6464, 400, _N2,
                      name="deg1", edge_split=True)
_DEG2 = _make_deg_agg(8, 512, 512, 640, 40, 576, 32, _N3, name="deg2")


@jax.jit
def kernel(x, src0, dst0, src1, dst1, src2, dst2, embed_W,
           Wself0, Wneigh0, b0, Wself1, Wneigh1, b1, Wself2, Wneigh2, b2,
           fcW, fcb):
    def _pad_edges(src, dst, total, dump):
        pad = total - src.shape[0]
        return (jnp.concatenate([src, jnp.zeros((pad,), jnp.int32)]),
                jnp.concatenate([dst, jnp.full((pad,), dump, jnp.int32)]))

    src0p, dst0p = _pad_edges(src0, dst0, 409600, _N1)
    src1p, dst1p = _pad_edges(src1, dst1, 114688, _N2)
    src2p, dst2p = _pad_edges(src2, dst2, 18432, _N3)

    d0 = _DEG0(dst0p)
    d1 = _DEG1(dst1p)
    d2 = _DEG2(dst2p)

    h0 = _embed(x, embed_W.T)

    s0 = _SUM0(h0, src0p, dst0p)
    h1 = _layer0(h0, s0, d0, Wself0.T, Wneigh0.T, b0.reshape(1, -1),
                 n_out=25088, blk=784)

    s1 = _SUM1(h1, src1p, dst1p)
    h2 = _layer1(h1, s1, d1, Wself1.T, Wneigh1.T, b1.reshape(1, -1),
                 n_out=_N2, blk=800)

    s2 = _SUM2(h2, src2p, dst2p)
    return _layer2_fc(h2, s2, d2, Wself2.T, Wneigh2.T, b2.reshape(1, -1),
                      fcW.T, fcb.reshape(1, -1), n_out=_N3)


# speedup vs baseline: 1.1827x; 1.0080x over previous
import functools

import jax
import jax.numpy as jnp
from jax import lax
from jax.experimental import pallas as pl
from jax.experimental.pallas import tpu as pltpu
from jax.experimental.pallas import tpu_sc as plsc

F32 = jnp.float32

_N0, _N1, _N2, _N3 = 100000, 25000, 6400, 1024
_E0, _E1, _E2 = 400000, 102400, 16384
_F_IN, _H, _C = 512, 128, 128

_HX = 144
_NC, _NS = 2, 16
_B = 128


def _mm_body(x_ref, w_ref, o_ref):
    o_ref[...] = jnp.dot(x_ref[...], w_ref[...], preferred_element_type=F32)


def _embed(x, w_t):
    blk = 2000
    grid = _N0 // blk
    return pl.pallas_call(
        _mm_body,
        grid=(grid,),
        in_specs=[
            pl.BlockSpec((blk, _F_IN), lambda i: (i, 0)),
            pl.BlockSpec((_F_IN, _H), lambda i: (0, 0)),
        ],
        out_specs=pl.BlockSpec((blk, _H), lambda i: (i, 0)),
        out_shape=jax.ShapeDtypeStruct((_N0, _H), F32),
    )(x, w_t)


def _layer0_body(hd_ref, sm_ref, dg_ref, ws_ref, wn_ref, b_ref, o_ref):
    deg = jnp.maximum(dg_ref[...][:, :1], 1.0)
    neigh = sm_ref[...] / deg
    acc = (jnp.dot(hd_ref[...], ws_ref[...], preferred_element_type=F32)
           + jnp.dot(neigh, wn_ref[...], preferred_element_type=F32)
           + b_ref[...])
    o_ref[...] = jnp.maximum(acc, 0.0)


def _layer12_body(hd_ref, sma_ref, smb_ref, dga_ref, dgb_ref,
                  ws_ref, wn_ref, b_ref, o_ref):
    dg = dga_ref[...] + dgb_ref[...]
    deg = jnp.maximum(dg[:, :1], 1.0)
    neigh = (sma_ref[...] + smb_ref[...]) / deg
    acc = (jnp.dot(hd_ref[...], ws_ref[...], preferred_element_type=F32)
           + jnp.dot(neigh, wn_ref[...], preferred_element_type=F32)
           + b_ref[...])
    o_ref[...] = jnp.maximum(acc, 0.0)


def _layer2_fc_body(hd_ref, sma_ref, smb_ref, dg_ref,
                    ws_ref, wn_ref, b_ref, fw_ref, fb_ref, o_ref):
    deg = jnp.maximum(dg_ref[...][:, :1], 1.0)
    neigh = (sma_ref[...] + smb_ref[...]) / deg
    acc = (jnp.dot(hd_ref[...], ws_ref[...], preferred_element_type=F32)
           + jnp.dot(neigh, wn_ref[...], preferred_element_type=F32)
           + b_ref[...])
    o_ref[...] = jnp.dot(acc, fw_ref[...], preferred_element_type=F32) + fb_ref[...]


def _layer0(h_prev, sums, deg, ws_t, wn_t, b, n_out, blk):
    grid = n_out // blk
    return pl.pallas_call(
        _layer0_body,
        grid=(grid,),
        in_specs=[
            pl.BlockSpec((blk, _H), lambda i: (i, 0)),
            pl.BlockSpec((blk, _H), lambda i: (i, 0)),
            pl.BlockSpec((blk, 16), lambda i: (i, 0)),
            pl.BlockSpec((_H, _H), lambda i: (0, 0)),
            pl.BlockSpec((_H, _H), lambda i: (0, 0)),
            pl.BlockSpec((1, _H), lambda i: (0, 0)),
        ],
        out_specs=pl.BlockSpec((blk, _H), lambda i: (i, 0)),
        out_shape=jax.ShapeDtypeStruct((n_out, _H), F32),
    )(h_prev, sums, deg, ws_t, wn_t, b)


def _layer1(h_prev, sums, deg, ws_t, wn_t, b, n_out, blk):
    grid = n_out // blk
    return pl.pallas_call(
        _layer12_body,
        grid=(grid,),
        in_specs=[
            pl.BlockSpec((blk, _H), lambda i: (i, 0)),
            pl.BlockSpec((blk, _H), lambda i: (i, 0)),
            pl.BlockSpec((blk, _H), lambda i: (i, 0)),
            pl.BlockSpec((blk, 16), lambda i: (i, 0)),
            pl.BlockSpec((blk, 16), lambda i: (i, 0)),
            pl.BlockSpec((_H, _H), lambda i: (0, 0)),
            pl.BlockSpec((_H, _H), lambda i: (0, 0)),
            pl.BlockSpec((1, _H), lambda i: (0, 0)),
        ],
        out_specs=pl.BlockSpec((blk, _H), lambda i: (i, 0)),
        out_shape=jax.ShapeDtypeStruct((n_out, _H), F32),
    )(h_prev, sums[:n_out], sums[n_out:], deg[:n_out], deg[n_out:],
      ws_t, wn_t, b)


def _layer2_fc(h_prev, sums, deg, ws_t, wn_t, b, fw_t, fb, n_out):
    return pl.pallas_call(
        _layer2_fc_body,
        grid=(1,),
        in_specs=[
            pl.BlockSpec((n_out, _H), lambda i: (0, 0)),
            pl.BlockSpec((n_out, _H), lambda i: (0, 0)),
            pl.BlockSpec((n_out, _H), lambda i: (0, 0)),
            pl.BlockSpec((n_out, 16), lambda i: (0, 0)),
            pl.BlockSpec((_H, _H), lambda i: (0, 0)),
            pl.BlockSpec((_H, _H), lambda i: (0, 0)),
            pl.BlockSpec((1, _H), lambda i: (0, 0)),
            pl.BlockSpec((_H, _C), lambda i: (0, 0)),
            pl.BlockSpec((1, _C), lambda i: (0, 0)),
        ],
        out_specs=pl.BlockSpec((n_out, _C), lambda i: (0, 0)),
        out_shape=jax.ShapeDtypeStruct((n_out, _C), F32),
    )(h_prev, sums[:n_out], sums[n_out:], deg, ws_t, wn_t, b, fw_t, fb)


_MESH = plsc.VectorSubcoreMesh(core_axis_name="c", subcore_axis_name="s",
                               num_cores=_NC, num_subcores=_NS)


def _make_sum_agg(chunks, split, rng, alloc, zspan, trash, wout, n_out,
                  B=96, nbuf=2, name="sum", W=_H, ncopy=1,
                  edge_split=False):
    assert chunks % nbuf == 0 and chunks >= 2 * nbuf
    assert zspan == ncopy * alloc // 16 and zspan % 8 == 0
    out_shape = (_NC * n_out, W) if edge_split else (n_out, W)

    @functools.partial(
        pl.kernel,
        out_type=jax.ShapeDtypeStruct(out_shape, F32),
        mesh=_MESH,
        name=name,
        compiler_params=pltpu.CompilerParams(
            use_tc_tiling_on_sc=(W == _H)),
        scratch_types=[
            pltpu.VMEM((nbuf, B), jnp.int32),
            pltpu.VMEM((nbuf, B), jnp.int32),
            pltpu.VMEM((nbuf, B), jnp.int32),
            pltpu.VMEM((nbuf, B, W), F32),
            pltpu.VMEM((2, wout if ncopy > 1 else 8, W), F32),
            pltpu.VMEM_SHARED((ncopy * alloc, W), F32),
        ] + [pltpu.SemaphoreType.DMA] * (3 * nbuf),
    )
    def agg(h_hbm, src_hbm, dst_hbm, sums_out,
            idx_src, idx_dst, idx_loc, rows, wr, sums_sh, *sems):
        gsem, ssem, isem = sems[:nbuf], sems[nbuf:2 * nbuf], sems[2 * nbuf:]
        c = lax.axis_index("c")
        s = lax.axis_index("s")

        def zrow(i, _):
            def zcol(j, _):
                rows[0, i, pl.ds(j * 16, 16)] = jnp.zeros((16,), F32)
                return 0
            lax.fori_loop(0, W // 16, zcol, 0)
            return 0
        lax.fori_loop(0, 8, zrow, 0)

        def zshared(t, _):
            off = s * zspan + t * 8
            pltpu.sync_copy(rows.at[0].at[pl.ds(0, 8)],
                            sums_sh.at[pl.ds(off, 8)])
            return 0
        lax.fori_loop(0, zspan // 8, zshared, 0)

        plsc.subcore_barrier()

        if edge_split:
            lo = 0
            base0 = (c * _NS + s) * chunks * B
        else:
            lo = c * split
            base0 = s * chunks * B
        cbase = lax.rem(s, ncopy) * alloc if ncopy > 1 else 0

        def compute_loc(b):
            def loc16(k, _):
                d = idx_dst[b, pl.ds(k * 16, 16)]
                l = d - lo
                ok = (l >= 0) & (l < rng)
                idx_loc[b, pl.ds(k * 16, 16)] = jnp.where(ok, l, trash) + cbase
                return 0
            lax.fori_loop(0, B // 16, loc16, 0)

        def fire_gather(b):
            pltpu.async_copy(h_hbm.at[idx_src.at[b]], rows.at[b], gsem[b])

        def wait_gather(b):
            pltpu.make_async_copy(h_hbm.at[idx_src.at[b]], rows.at[b],
                                  gsem[b]).wait()

        def fire_scatter(b):
            pltpu.async_copy(rows.at[b], sums_sh.at[idx_loc.at[b]], ssem[b],
                             add=True)

        def wait_scatter(b):
            pltpu.make_async_copy(rows.at[b], sums_sh.at[idx_loc.at[b]],
                                  ssem[b]).wait()

        for b in range(nbuf):
            base = base0 + b * B
            pltpu.sync_copy(src_hbm.at[pl.ds(base, B)], idx_src.at[b])
            pltpu.sync_copy(dst_hbm.at[pl.ds(base, B)], idx_dst.at[b])
            fire_gather(b)

        def body(g, _):
            for b in range(nbuf):
                nbase = base0 + (nbuf * g + b + nbuf) * B
                wait_gather(b)
                compute_loc(b)
                fire_scatter(b)
                pltpu.async_copy(src_hbm.at[pl.ds(nbase, B)],
                                 idx_src.at[b], isem[b])
                pltpu.async_copy(dst_hbm.at[pl.ds(nbase, B)],
                                 idx_dst.at[b], isem[b])
                wait_scatter(b)
                pltpu.make_async_copy(src_hbm.at[pl.ds(nbase, B)],
                                      idx_src.at[b], isem[b]).wait()
                pltpu.make_async_copy(dst_hbm.at[pl.ds(nbase, B)],
                                      idx_dst.at[b], isem[b]).wait()
                fire_gather(b)
            return 0
        lax.fori_loop(0, chunks // nbuf - 1, body, 0)

        for b in range(nbuf):
            wait_gather(b)
            compute_loc(b)
            fire_scatter(b)
            wait_scatter(b)

        plsc.subcore_barrier()

        if edge_split:
            dst_ref = sums_out.at[pl.ds(c * n_out + s * wout, wout)]
        else:
            dst_ref = sums_out.at[pl.ds(c * split + s * wout, wout)]
        if ncopy == 1:
            pltpu.sync_copy(sums_sh.at[pl.ds(s * wout, wout)], dst_ref)
        else:
            pltpu.sync_copy(sums_sh.at[pl.ds(s * wout, wout)], wr.at[0])
            for k in range(1, ncopy):
                pltpu.sync_copy(sums_sh.at[pl.ds(k * alloc + s * wout, wout)],
                                wr.at[1])
                def radd(i, _):
                    def cadd(j, _):
                        wr[0, i, pl.ds(j * 16, 16)] = (
                            wr[0, i, pl.ds(j * 16, 16)]
                            + wr[1, i, pl.ds(j * 16, 16)])
                        return 0
                    lax.fori_loop(0, W // 16, cadd, 0)
                    return 0
                lax.fori_loop(0, wout, radd, 0)
            pltpu.sync_copy(wr.at[0], dst_ref)

    return agg


def _make_deg_agg(chunks, split, rng, alloc, zspan, trash, wout, n_out,
                  name="deg", edge_split=False, nbuf=4):
    assert chunks % nbuf == 0 and chunks >= 2 * nbuf
    out_shape = (_NC * n_out, 16) if edge_split else (n_out, 16)

    @functools.partial(
        pl.kernel,
        out_type=jax.ShapeDtypeStruct(out_shape, F32),
        mesh=_MESH,
        name=name,
        scratch_types=[
            pltpu.VMEM((nbuf, _B), jnp.int32),
            pltpu.VMEM((nbuf, _B), jnp.int32),
            pltpu.VMEM((_B, 16), F32),
            pltpu.VMEM_SHARED((alloc, 16), F32),
        ] + [pltpu.SemaphoreType.DMA] * (2 * nbuf),
    )
    def agg(dst_hbm, deg_out, idx_dst, idx_loc, ones_b, deg_sh, *sems):
        ssem, isem = sems[:nbuf], sems[nbuf:]
        c = lax.axis_index("c")
        s = lax.axis_index("s")

        def zrow(i, _):
            ones_b[i, :] = jnp.zeros((16,), F32)
            return 0
        lax.fori_loop(0, 8, zrow, 0)

        def zshared(t, _):
            off = s * zspan + t * 8
            pltpu.sync_copy(ones_b.at[pl.ds(0, 8)], deg_sh.at[pl.ds(off, 8)])
            return 0
        lax.fori_loop(0, zspan // 8, zshared, 0)

        def fill_ones(i, _):
            ones_b[i, :] = jnp.ones((16,), F32)
            return 0
        lax.fori_loop(0, _B, fill_ones, 0)

        plsc.subcore_barrier()

        if edge_split:
            lo = 0
            base0 = (c * _NS + s) * chunks * _B
        else:
            lo = c * split
            base0 = s * chunks * _B

        def compute_loc(b):
            def loc16(k, _):
                d = idx_dst[b, pl.ds(k * 16, 16)]
                l = d - lo
                ok = (l >= 0) & (l < rng)
                idx_loc[b, pl.ds(k * 16, 16)] = jnp.where(ok, l, trash)
                return 0
            lax.fori_loop(0, _B // 16, loc16, 0)

        def fire_scatter(b):
            pltpu.async_copy(ones_b, deg_sh.at[idx_loc.at[b]], ssem[b],
                             add=True)

        def wait_scatter(b):
            pltpu.make_async_copy(ones_b, deg_sh.at[idx_loc.at[b]],
                                  ssem[b]).wait()

        for b in range(nbuf):
            base = base0 + b * _B
            pltpu.sync_copy(dst_hbm.at[pl.ds(base, _B)], idx_dst.at[b])
            compute_loc(b)
            fire_scatter(b)

        def body(g, _):
            for b in range(nbuf):
                nbase = base0 + (nbuf * g + b + nbuf) * _B
                pltpu.async_copy(dst_hbm.at[pl.ds(nbase, _B)],
                                 idx_dst.at[b], isem[b])
                wait_scatter(b)
                pltpu.make_async_copy(dst_hbm.at[pl.ds(nbase, _B)],
                                      idx_dst.at[b], isem[b]).wait()
                compute_loc(b)
                fire_scatter(b)
            return 0
        lax.fori_loop(0, chunks // nbuf - 1, body, 0)

        for b in range(nbuf):
            wait_scatter(b)

        plsc.subcore_barrier()

        if edge_split:
            off = c * n_out + s * wout
        else:
            off = c * split + s * wout
        pltpu.sync_copy(deg_sh.at[pl.ds(s * wout, wout)],
                        deg_out.at[pl.ds(off, wout)])

    return agg


_SUM0 = _make_sum_agg(262, 12544, 12544, 12672, 792, 12600, 784, 25088,
                      name="sum0")
_DEG0 = _make_deg_agg(196, 12544, 12544, 12672, 792, 12600, 784, 25088,
                      name="deg0")
_SUM1 = _make_sum_agg(34, 0, _N2, 6528, 408, 6464, 400, _N2,
                      name="sum1", edge_split=True)
_SUM2 = _make_sum_agg(6, 0, _N3, 1152, 288, 1088, 64, _N3,
                      name="sum2", ncopy=4, edge_split=True)
_DEG1 = _make_deg_agg(26, 0, _N2, 6528, 408, 6464, 400, _N2,
                      name="deg1", edge_split=True, nbuf=2)
_DEG2 = _make_deg_agg(8, 512, 512, 640, 40, 576, 32, _N3, name="deg2")


@jax.jit
def kernel(x, src0, dst0, src1, dst1, src2, dst2, embed_W,
           Wself0, Wneigh0, b0, Wself1, Wneigh1, b1, Wself2, Wneigh2, b2,
           fcW, fcb):
    def _pad_tiled(arr, blocks, per_blk, fill):
        real = arr.reshape(blocks, -1)
        pad = jnp.full((blocks, per_blk - real.shape[1]), fill, jnp.int32)
        return jnp.concatenate([real, pad], axis=1).reshape(-1)

    src0s = _pad_tiled(src0, 16, 25152, 0)
    dst0s = _pad_tiled(dst0, 16, 25152, _N1)
    dst0d = _pad_tiled(dst0, 16, 25088, _N1)
    src1s = _pad_tiled(src1, 32, 3264, 0)
    dst1s = _pad_tiled(dst1, 32, 3264, _N2)
    dst1d = _pad_tiled(dst1, 32, 3328, _N2)
    src2s = _pad_tiled(src2, 32, 576, 0)
    dst2s = _pad_tiled(dst2, 32, 576, _N3)

    d0 = _DEG0(dst0d)
    d1 = _DEG1(dst1d)
    d2 = _DEG2(dst2)

    h0 = _embed(x, embed_W.T)

    s0 = _SUM0(h0, src0s, dst0s)
    h1 = _layer0(h0, s0, d0, Wself0.T, Wneigh0.T, b0.reshape(1, -1),
                 n_out=25088, blk=784)

    s1 = _SUM1(h1, src1s, dst1s)
    h2 = _layer1(h1, s1, d1, Wself1.T, Wneigh1.T, b1.reshape(1, -1),
                 n_out=_N2, blk=800)

    s2 = _SUM2(h2, src2s, dst2s)
    return _layer2_fc(h2, s2, d2, Wself2.T, Wneigh2.T, b2.reshape(1, -1),
                      fcW.T, fcb.reshape(1, -1), n_out=_N3)


# speedup vs baseline: 1.2890x; 1.0899x over previous
import functools

import jax
import jax.numpy as jnp
from jax import lax
from jax.experimental import pallas as pl
from jax.experimental.pallas import tpu as pltpu
from jax.experimental.pallas import tpu_sc as plsc

F32 = jnp.float32

_N0, _N1, _N2, _N3 = 100000, 25000, 6400, 1024
_E0, _E1, _E2 = 400000, 102400, 16384
_F_IN, _H, _C = 512, 128, 128

_HX = 144
_NC, _NS = 2, 16
_B = 128


def _mm_body(x_ref, w_ref, o_ref):
    o_ref[...] = jnp.dot(x_ref[...], w_ref[...], preferred_element_type=F32)


def _embed(x, w_t):
    blk = 2000
    grid = _N0 // blk
    return pl.pallas_call(
        _mm_body,
        grid=(grid,),
        in_specs=[
            pl.BlockSpec((blk, _F_IN), lambda i: (i, 0)),
            pl.BlockSpec((_F_IN, _H), lambda i: (0, 0)),
        ],
        out_specs=pl.BlockSpec((blk, _H), lambda i: (i, 0)),
        out_shape=jax.ShapeDtypeStruct((_N0, _H), F32),
    )(x, w_t)


def _layer0_body(hd_ref, sm_ref, dg_ref, ws_ref, wn_ref, b_ref, o_ref):
    deg = jnp.maximum(dg_ref[...][:, :1], 1.0)
    neigh = sm_ref[...] / deg
    acc = (jnp.dot(hd_ref[...], ws_ref[...], preferred_element_type=F32)
           + jnp.dot(neigh, wn_ref[...], preferred_element_type=F32)
           + b_ref[...])
    o_ref[...] = jnp.maximum(acc, 0.0)


def _layer12_body(hd_ref, sma_ref, smb_ref, dga_ref, dgb_ref,
                  ws_ref, wn_ref, b_ref, o_ref):
    dg = dga_ref[...] + dgb_ref[...]
    deg = jnp.maximum(dg[:, :1], 1.0)
    neigh = (sma_ref[...] + smb_ref[...]) / deg
    acc = (jnp.dot(hd_ref[...], ws_ref[...], preferred_element_type=F32)
           + jnp.dot(neigh, wn_ref[...], preferred_element_type=F32)
           + b_ref[...])
    o_ref[...] = jnp.maximum(acc, 0.0)


def _layer2_fc_body(hd_ref, sma_ref, smb_ref, dg_ref,
                    ws_ref, wn_ref, b_ref, fw_ref, fb_ref, o_ref):
    deg = jnp.maximum(dg_ref[...][:, :1], 1.0)
    neigh = (sma_ref[...] + smb_ref[...]) / deg
    acc = (jnp.dot(hd_ref[...], ws_ref[...], preferred_element_type=F32)
           + jnp.dot(neigh, wn_ref[...], preferred_element_type=F32)
           + b_ref[...])
    o_ref[...] = jnp.dot(acc, fw_ref[...], preferred_element_type=F32) + fb_ref[...]


def _layer0(h_prev, sums, deg, ws_t, wn_t, b, n_out, blk):
    grid = n_out // blk
    return pl.pallas_call(
        _layer0_body,
        grid=(grid,),
        in_specs=[
            pl.BlockSpec((blk, _H), lambda i: (i, 0)),
            pl.BlockSpec((blk, _H), lambda i: (i, 0)),
            pl.BlockSpec((blk, 16), lambda i: (i, 0)),
            pl.BlockSpec((_H, _H), lambda i: (0, 0)),
            pl.BlockSpec((_H, _H), lambda i: (0, 0)),
            pl.BlockSpec((1, _H), lambda i: (0, 0)),
        ],
        out_specs=pl.BlockSpec((blk, _H), lambda i: (i, 0)),
        out_shape=jax.ShapeDtypeStruct((n_out, _H), F32),
    )(h_prev, sums, deg, ws_t, wn_t, b)


def _layer1(h_prev, sums, deg, ws_t, wn_t, b, n_out, blk):
    grid = n_out // blk
    return pl.pallas_call(
        _layer12_body,
        grid=(grid,),
        in_specs=[
            pl.BlockSpec((blk, _H), lambda i: (i, 0)),
            pl.BlockSpec((blk, _H), lambda i: (i, 0)),
            pl.BlockSpec((blk, _H), lambda i: (i, 0)),
            pl.BlockSpec((blk, 16), lambda i: (i, 0)),
            pl.BlockSpec((blk, 16), lambda i: (i, 0)),
            pl.BlockSpec((_H, _H), lambda i: (0, 0)),
            pl.BlockSpec((_H, _H), lambda i: (0, 0)),
            pl.BlockSpec((1, _H), lambda i: (0, 0)),
        ],
        out_specs=pl.BlockSpec((blk, _H), lambda i: (i, 0)),
        out_shape=jax.ShapeDtypeStruct((n_out, _H), F32),
    )(h_prev, sums[:n_out], sums[n_out:], deg[:n_out], deg[n_out:],
      ws_t, wn_t, b)


def _layer2_fc(h_prev, sums, deg, ws_t, wn_t, b, fw_t, fb, n_out):
    return pl.pallas_call(
        _layer2_fc_body,
        grid=(1,),
        in_specs=[
            pl.BlockSpec((n_out, _H), lambda i: (0, 0)),
            pl.BlockSpec((n_out, _H), lambda i: (0, 0)),
            pl.BlockSpec((n_out, _H), lambda i: (0, 0)),
            pl.BlockSpec((n_out, 16), lambda i: (0, 0)),
            pl.BlockSpec((_H, _H), lambda i: (0, 0)),
            pl.BlockSpec((_H, _H), lambda i: (0, 0)),
            pl.BlockSpec((1, _H), lambda i: (0, 0)),
            pl.BlockSpec((_H, _C), lambda i: (0, 0)),
            pl.BlockSpec((1, _C), lambda i: (0, 0)),
        ],
        out_specs=pl.BlockSpec((n_out, _C), lambda i: (0, 0)),
        out_shape=jax.ShapeDtypeStruct((n_out, _C), F32),
    )(h_prev, sums[:n_out], sums[n_out:], deg, ws_t, wn_t, b, fw_t, fb)


_MESH = plsc.VectorSubcoreMesh(core_axis_name="c", subcore_axis_name="s",
                               num_cores=_NC, num_subcores=_NS)


def _make_sum_agg(chunks, split, rng, alloc, zspan, trash, wout, n_out,
                  B=96, nbuf=2, name="sum", W=_H, ncopy=1,
                  edge_split=False):
    assert chunks % nbuf == 0 and chunks >= 2 * nbuf
    assert zspan == ncopy * alloc // 16 and zspan % 8 == 0
    out_shape = (_NC * n_out, W) if edge_split else (n_out, W)

    @functools.partial(
        pl.kernel,
        out_type=jax.ShapeDtypeStruct(out_shape, F32),
        mesh=_MESH,
        name=name,
        compiler_params=pltpu.CompilerParams(
            use_tc_tiling_on_sc=(W == _H)),
        scratch_types=[
            pltpu.VMEM((nbuf, B), jnp.int32),
            pltpu.VMEM((nbuf, B), jnp.int32),
            pltpu.VMEM((nbuf, B), jnp.int32),
            pltpu.VMEM((nbuf, B, W), F32),
            pltpu.VMEM((2, wout if ncopy > 1 else 8, W), F32),
            pltpu.VMEM_SHARED((ncopy * alloc, W), F32),
        ] + [pltpu.SemaphoreType.DMA] * (3 * nbuf),
    )
    def agg(h_hbm, src_hbm, dst_hbm, sums_out,
            idx_src, idx_dst, idx_loc, rows, wr, sums_sh, *sems):
        gsem, ssem, isem = sems[:nbuf], sems[nbuf:2 * nbuf], sems[2 * nbuf:]
        c = lax.axis_index("c")
        s = lax.axis_index("s")

        def zrow(i, _):
            def zcol(j, _):
                rows[0, i, pl.ds(j * 16, 16)] = jnp.zeros((16,), F32)
                return 0
            lax.fori_loop(0, W // 16, zcol, 0)
            return 0
        lax.fori_loop(0, 8, zrow, 0)

        def zshared(t, _):
            off = s * zspan + t * 8
            pltpu.sync_copy(rows.at[0].at[pl.ds(0, 8)],
                            sums_sh.at[pl.ds(off, 8)])
            return 0
        lax.fori_loop(0, zspan // 8, zshared, 0)

        plsc.subcore_barrier()

        if edge_split:
            lo = 0
            base0 = (c * _NS + s) * chunks * B
        else:
            lo = c * split
            base0 = s * chunks * B
        cbase = lax.rem(s, ncopy) * alloc if ncopy > 1 else 0

        def compute_loc(b):
            def loc16(k, _):
                d = idx_dst[b, pl.ds(k * 16, 16)]
                l = d - lo
                ok = (l >= 0) & (l < rng)
                idx_loc[b, pl.ds(k * 16, 16)] = jnp.where(ok, l, trash) + cbase
                return 0
            lax.fori_loop(0, B // 16, loc16, 0)

        def fire_gather(b):
            pltpu.async_copy(h_hbm.at[idx_src.at[b]], rows.at[b], gsem[b])

        def wait_gather(b):
            pltpu.make_async_copy(h_hbm.at[idx_src.at[b]], rows.at[b],
                                  gsem[b]).wait()

        def fire_scatter(b):
            pltpu.async_copy(rows.at[b], sums_sh.at[idx_loc.at[b]], ssem[b],
                             add=True)

        def wait_scatter(b):
            pltpu.make_async_copy(rows.at[b], sums_sh.at[idx_loc.at[b]],
                                  ssem[b]).wait()

        for b in range(nbuf):
            base = base0 + b * B
            pltpu.sync_copy(src_hbm.at[pl.ds(base, B)], idx_src.at[b])
            pltpu.sync_copy(dst_hbm.at[pl.ds(base, B)], idx_dst.at[b])
            fire_gather(b)

        def body(g, _):
            for b in range(nbuf):
                nbase = base0 + (nbuf * g + b + nbuf) * B
                wait_gather(b)
                compute_loc(b)
                fire_scatter(b)
                pltpu.async_copy(src_hbm.at[pl.ds(nbase, B)],
                                 idx_src.at[b], isem[b])
                pltpu.async_copy(dst_hbm.at[pl.ds(nbase, B)],
                                 idx_dst.at[b], isem[b])
                wait_scatter(b)
                pltpu.make_async_copy(src_hbm.at[pl.ds(nbase, B)],
                                      idx_src.at[b], isem[b]).wait()
                pltpu.make_async_copy(dst_hbm.at[pl.ds(nbase, B)],
                                      idx_dst.at[b], isem[b]).wait()
                fire_gather(b)
            return 0
        lax.fori_loop(0, chunks // nbuf - 1, body, 0)

        for b in range(nbuf):
            wait_gather(b)
            compute_loc(b)
            fire_scatter(b)
            wait_scatter(b)

        plsc.subcore_barrier()

        if edge_split:
            dst_ref = sums_out.at[pl.ds(c * n_out + s * wout, wout)]
        else:
            dst_ref = sums_out.at[pl.ds(c * split + s * wout, wout)]
        if ncopy == 1:
            pltpu.sync_copy(sums_sh.at[pl.ds(s * wout, wout)], dst_ref)
        else:
            pltpu.sync_copy(sums_sh.at[pl.ds(s * wout, wout)], wr.at[0])
            for k in range(1, ncopy):
                pltpu.sync_copy(sums_sh.at[pl.ds(k * alloc + s * wout, wout)],
                                wr.at[1])
                def radd(i, _):
                    def cadd(j, _):
                        wr[0, i, pl.ds(j * 16, 16)] = (
                            wr[0, i, pl.ds(j * 16, 16)]
                            + wr[1, i, pl.ds(j * 16, 16)])
                        return 0
                    lax.fori_loop(0, W // 16, cadd, 0)
                    return 0
                lax.fori_loop(0, wout, radd, 0)
            pltpu.sync_copy(wr.at[0], dst_ref)

    return agg


def _make_deg_agg(chunks, split, rng, alloc, zspan, trash, wout, n_out,
                  name="deg", edge_split=False, nbuf=4):
    assert chunks % nbuf == 0 and chunks >= 2 * nbuf
    out_shape = (_NC * n_out, 16) if edge_split else (n_out, 16)

    @functools.partial(
        pl.kernel,
        out_type=jax.ShapeDtypeStruct(out_shape, F32),
        mesh=_MESH,
        name=name,
        scratch_types=[
            pltpu.VMEM((nbuf, _B), jnp.int32),
            pltpu.VMEM((nbuf, _B), jnp.int32),
            pltpu.VMEM((_B, 16), F32),
            pltpu.VMEM_SHARED((alloc, 16), F32),
        ] + [pltpu.SemaphoreType.DMA] * (2 * nbuf),
    )
    def agg(dst_hbm, deg_out, idx_dst, idx_loc, ones_b, deg_sh, *sems):
        ssem, isem = sems[:nbuf], sems[nbuf:]
        c = lax.axis_index("c")
        s = lax.axis_index("s")

        def zrow(i, _):
            ones_b[i, :] = jnp.zeros((16,), F32)
            return 0
        lax.fori_loop(0, 8, zrow, 0)

        def zshared(t, _):
            off = s * zspan + t * 8
            pltpu.sync_copy(ones_b.at[pl.ds(0, 8)], deg_sh.at[pl.ds(off, 8)])
            return 0
        lax.fori_loop(0, zspan // 8, zshared, 0)

        def fill_ones(i, _):
            ones_b[i, :] = jnp.ones((16,), F32)
            return 0
        lax.fori_loop(0, _B, fill_ones, 0)

        plsc.subcore_barrier()

        if edge_split:
            lo = 0
            base0 = (c * _NS + s) * chunks * _B
        else:
            lo = c * split
            base0 = s * chunks * _B

        def compute_loc(b):
            def loc16(k, _):
                d = idx_dst[b, pl.ds(k * 16, 16)]
                l = d - lo
                ok = (l >= 0) & (l < rng)
                idx_loc[b, pl.ds(k * 16, 16)] = jnp.where(ok, l, trash)
                return 0
            lax.fori_loop(0, _B // 16, loc16, 0)

        def fire_scatter(b):
            pltpu.async_copy(ones_b, deg_sh.at[idx_loc.at[b]], ssem[b],
                             add=True)

        def wait_scatter(b):
            pltpu.make_async_copy(ones_b, deg_sh.at[idx_loc.at[b]],
                                  ssem[b]).wait()

        for b in range(nbuf):
            base = base0 + b * _B
            pltpu.sync_copy(dst_hbm.at[pl.ds(base, _B)], idx_dst.at[b])
            compute_loc(b)
            fire_scatter(b)

        def body(g, _):
            for b in range(nbuf):
                nbase = base0 + (nbuf * g + b + nbuf) * _B
                pltpu.async_copy(dst_hbm.at[pl.ds(nbase, _B)],
                                 idx_dst.at[b], isem[b])
                wait_scatter(b)
                pltpu.make_async_copy(dst_hbm.at[pl.ds(nbase, _B)],
                                      idx_dst.at[b], isem[b]).wait()
                compute_loc(b)
                fire_scatter(b)
            return 0
        lax.fori_loop(0, chunks // nbuf - 1, body, 0)

        for b in range(nbuf):
            wait_scatter(b)

        plsc.subcore_barrier()

        if edge_split:
            off = c * n_out + s * wout
        else:
            off = c * split + s * wout
        pltpu.sync_copy(deg_sh.at[pl.ds(s * wout, wout)],
                        deg_out.at[pl.ds(off, wout)])

    return agg


_SUM0 = _make_sum_agg(262, 12544, 12544, 12672, 792, 12600, 784, 25088,
                      name="sum0")
_DEG0 = _make_deg_agg(196, 12544, 12544, 12672, 792, 12600, 784, 25088,
                      name="deg0")
_SUM1 = _make_sum_agg(34, 0, _N2, 6528, 408, 6464, 400, _N2,
                      name="sum1", edge_split=True)
_SUM2 = _make_sum_agg(6, 0, _N3, 1152, 288, 1088, 64, _N3,
                      name="sum2", ncopy=4, edge_split=True)
_DEG1 = _make_deg_agg(26, 0, _N2, 6528, 408, 6464, 400, _N2,
                      name="deg1", edge_split=True, nbuf=2)
_DEG2 = _make_deg_agg(8, 512, 512, 640, 40, 576, 32, _N3, name="deg2")


@jax.jit
def kernel(x, src0, dst0, src1, dst1, src2, dst2, embed_W,
           Wself0, Wneigh0, b0, Wself1, Wneigh1, b1, Wself2, Wneigh2, b2,
           fcW, fcb):
    def _pad_tiled(arr, blocks, per_blk, fill):
        real = arr.reshape(blocks, -1)
        pad = jnp.full((blocks, per_blk - real.shape[1]), fill, jnp.int32)
        return jnp.concatenate([real, pad], axis=1).reshape(-1)

    src0s = _pad_tiled(src0, 16, 25152, 0)
    dst0s = _pad_tiled(dst0, 16, 25152, _N1)
    dst0d = _pad_tiled(dst0, 16, 25088, _N1)
    src1s = _pad_tiled(src1, 32, 3264, 0)
    dst1s = _pad_tiled(dst1, 32, 3264, _N2)
    dst1d = _pad_tiled(dst1, 32, 3328, _N2)
    src2s = _pad_tiled(src2, 32, 576, 0)
    dst2s = _pad_tiled(dst2, 32, 576, _N3)

    d0 = _DEG0(dst0d)
    d1 = _DEG1(dst1d)
    d2 = _DEG2(dst2)

    h0 = _embed(x, embed_W.T)

    src0s = src0s + jnp.minimum(d0[0, 0].astype(jnp.int32), 0)
    s0 = _SUM0(h0, src0s, dst0s)
    h1 = _layer0(h0, s0, d0, Wself0.T, Wneigh0.T, b0.reshape(1, -1),
                 n_out=25088, blk=784)

    s1 = _SUM1(h1, src1s, dst1s)
    h2 = _layer1(h1, s1, d1, Wself1.T, Wneigh1.T, b1.reshape(1, -1),
                 n_out=_N2, blk=800)

    s2 = _SUM2(h2, src2s, dst2s)
    return _layer2_fc(h2, s2, d2, Wself2.T, Wneigh2.T, b2.reshape(1, -1),
                      fcW.T, fcb.reshape(1, -1), n_out=_N3)
